# baseline (speedup 1.0000x reference)
import functools

import jax
import jax.numpy as jnp
from jax import lax
from jax.experimental import pallas as pl
from jax.experimental.pallas import tpu as pltpu
from jax.experimental.pallas import tpu_sc as plsc

N_CHEM = 10000
N_GENE = 10000
E = 320000
D = 128
CHUNK = 64
NSUB = 16
NCHUNKS = 5024
N_PER_TILE = NCHUNKS // NSUB
EPAD = NCHUNKS * CHUNK
NPAD = 10000
ZONE = 632
LAST_ZONE = NPAD - 15 * ZONE
VALID_GATE = 50
GATE_ROWS = 64
NBLK = 10
BLK = N_CHEM // NBLK


def _sigmoid(x):
    return 1.0 / (1.0 + jnp.exp(-x))


def _prep_body(xc, xg, wscg, bscg, wdcg, bdcg, wsgc, bsgc, wdgc, bdgc,
               cat, wccg, bccg, wcgc, bcgc,
               xs_cg, xd_cg, xs_gc, xd_gc, g_cg, g_gc):
    c = xc[...]
    g = xg[...]
    xs_cg[...] = jnp.dot(c, wscg[...], preferred_element_type=jnp.float32) + bscg[...]
    xd_cg[...] = jnp.dot(g, wdcg[...], preferred_element_type=jnp.float32) + bdcg[...]
    xs_gc[...] = jnp.dot(g, wsgc[...], preferred_element_type=jnp.float32) + bsgc[...]
    xd_gc[...] = jnp.dot(c, wdgc[...], preferred_element_type=jnp.float32) + bdgc[...]

    @pl.when(pl.program_id(0) == 0)
    def _():
        t = cat[...]
        row = lax.broadcasted_iota(jnp.int32, (GATE_ROWS, D), 0)
        valid = row < VALID_GATE
        g_cg[...] = jnp.where(valid, _sigmoid(
            jnp.dot(t, wccg[...], preferred_element_type=jnp.float32) + bccg[...]) * 0.25, 0.0)
        g_gc[...] = jnp.where(valid, _sigmoid(
            jnp.dot(t, wcgc[...], preferred_element_type=jnp.float32) + bcgc[...]) * 0.25, 0.0)


def _out_body(ac, woc, boc, ag, wog, bog, oc, og):
    oc[...] = jnp.dot(ac[...], woc[...], preferred_element_type=jnp.float32) + boc[...]
    og[...] = jnp.dot(ag[...], wog[...], preferred_element_type=jnp.float32) + bog[...]


def _sc_body(xs0, xd0, g0, idx0h,
             xs1, xd1, g1, idx1h,
             out_g, out_c,
             idx_a, idx_b, di_sc_a, di_sc_b,
             src_a, dst_a, gate_a, src_b, dst_b, gate_b,
             acc, isem_a, isem_b, gsem_a, gsem_b, ssem_a, ssem_b):
    c = lax.axis_index("c")
    s = lax.axis_index("s")

    def _z(e, carry):
        for t in range(D // 16):
            src_a[e, pl.ds(t * 16, 16)] = jnp.zeros((16,), jnp.float32)
        return carry
    lax.fori_loop(0, CHUNK, _z, 0)
    for k in range(8):
        pltpu.sync_copy(src_a, acc.at[pl.ds(s * ZONE + k * CHUNK, CHUNK)])

    @pl.when(s < NSUB - 1)
    def _():
        pltpu.sync_copy(src_a, acc.at[pl.ds(s * ZONE + 8 * CHUNK, CHUNK)])
        pltpu.sync_copy(src_a.at[pl.ds(0, ZONE - 9 * CHUNK)],
                        acc.at[pl.ds(s * ZONE + 9 * CHUNK, ZONE - 9 * CHUNK)])

    @pl.when(s == NSUB - 1)
    def _():
        pltpu.sync_copy(src_a.at[pl.ds(0, LAST_ZONE - 8 * CHUNK)],
                        acc.at[pl.ds(s * ZONE + 8 * CHUNK, LAST_ZONE - 8 * CHUNK)])
    plsc.subcore_barrier()

    def _process(xs, xd, gt, idxh):
        n = N_PER_TILE
        sets = (
            (idx_a, di_sc_a, src_a, dst_a, gate_a, isem_a, gsem_a, ssem_a),
            (idx_b, di_sc_b, src_b, dst_b, gate_b, isem_b, gsem_b, ssem_b),
        )

        def fire_idx(j, st):
            idx, _, _, _, _, isem, _, _ = st
            pltpu.async_copy(idxh.at[s + NSUB * j], idx, isem)

        def gather_stage(j, st):
            idx, di_sc, src, dst, gate, isem, gsem, ssem = st

            @pl.when(j >= 2)
            def _():
                pltpu.make_async_copy(src, acc.at[di_sc], ssem).wait()

            pltpu.make_async_copy(idxh.at[s + NSUB * j], idx, isem).wait()
            pltpu.async_copy(xs.at[idx.at[0]], src, gsem)
            pltpu.async_copy(xd.at[idx.at[1]], dst, gsem)
            pltpu.async_copy(gt.at[idx.at[2]], gate, gsem)

        def compute_stage(j, st_a, st_b):
            idx, di_sc, src, dst, gate, isem, gsem, ssem = st_a
            pltpu.make_async_copy(xs.at[idx.at[0]], src, gsem).wait()
            pltpu.make_async_copy(xd.at[idx.at[1]], dst, gsem).wait()
            pltpu.make_async_copy(gt.at[idx.at[2]], gate, gsem).wait()

            for k in range(CHUNK // 16):
                sl = pl.ds(k * 16, 16)
                di_sc[sl] = idx[1, sl]

            @pl.when(j < n - 1)
            def _():
                fire_idx(j + 1, st_a)

            @plsc.parallel_loop(0, CHUNK, 1, unroll=2)
            def _(e):
                for t in range(D // 16):
                    sl = pl.ds(t * 16, 16)
                    src[e, sl] = (src[e, sl] + dst[e, sl]) * gate[e, sl]

            pltpu.async_copy(src, acc.at[di_sc], ssem, add=True)

        fire_idx(0, sets[0])
        fire_idx(1, sets[1])

        def iter_body(j, carry):
            @pl.when(j % 2 == 0)
            def _():
                @pl.when(j < n)
                def _():
                    gather_stage(j, sets[0])

                @pl.when(j >= 1)
                def _():
                    compute_stage(j, sets[1], sets[0])

            @pl.when(j % 2 == 1)
            def _():
                @pl.when(j < n)
                def _():
                    gather_stage(j, sets[1])
                compute_stage(j, sets[0], sets[1])
            return carry
        lax.fori_loop(0, n + 1, iter_body, 0)

        pltpu.make_async_copy(src_a, acc.at[di_sc_a], ssem_a).wait()
        pltpu.make_async_copy(src_b, acc.at[di_sc_b], ssem_b).wait()

    @pl.when(c == 0)
    def _():
        _process(xs0, xd0, g0, idx0h)

    @pl.when(c == 1)
    def _():
        _process(xs1, xd1, g1, idx1h)

    plsc.subcore_barrier()

    def _copy_out(dst):
        @pl.when(s < NSUB - 1)
        def _():
            pltpu.sync_copy(acc.at[pl.ds(s * ZONE, ZONE)],
                            dst.at[pl.ds(s * ZONE, ZONE)])

        @pl.when(s == NSUB - 1)
        def _():
            pltpu.sync_copy(acc.at[pl.ds(s * ZONE, LAST_ZONE)],
                            dst.at[pl.ds(s * ZONE, LAST_ZONE)])

    @pl.when(c == 0)
    def _():
        _copy_out(out_g)

    @pl.when(c == 1)
    def _():
        _copy_out(out_c)


_full128 = pl.BlockSpec((D, D), lambda i: (0, 0))
_full1x = pl.BlockSpec((1, D), lambda i: (0, 0))
_blk = pl.BlockSpec((BLK, D), lambda i: (i, 0))

_prep_call = pl.pallas_call(
    _prep_body,
    grid=(NBLK,),
    in_specs=[
        _blk, _blk,
        _full128, _full1x, _full128, _full1x,
        _full128, _full1x, _full128, _full1x,
        pl.BlockSpec((GATE_ROWS, GATE_ROWS), lambda i: (0, 0)),
        pl.BlockSpec((GATE_ROWS, D), lambda i: (0, 0)), _full1x,
        pl.BlockSpec((GATE_ROWS, D), lambda i: (0, 0)), _full1x,
    ],
    out_specs=[
        _blk, _blk, _blk, _blk,
        pl.BlockSpec((GATE_ROWS, D), lambda i: (0, 0)),
        pl.BlockSpec((GATE_ROWS, D), lambda i: (0, 0)),
    ],
    out_shape=[
        jax.ShapeDtypeStruct((N_CHEM, D), jnp.float32),
        jax.ShapeDtypeStruct((N_GENE, D), jnp.float32),
        jax.ShapeDtypeStruct((N_GENE, D), jnp.float32),
        jax.ShapeDtypeStruct((N_CHEM, D), jnp.float32),
        jax.ShapeDtypeStruct((GATE_ROWS, D), jnp.float32),
        jax.ShapeDtypeStruct((GATE_ROWS, D), jnp.float32),
    ],
)

_out_call = pl.pallas_call(
    _out_body,
    grid=(NBLK,),
    in_specs=[_blk, _full128, _full1x, _blk, _full128, _full1x],
    out_specs=[_blk, _blk],
    out_shape=[
        jax.ShapeDtypeStruct((N_CHEM, D), jnp.float32),
        jax.ShapeDtypeStruct((N_GENE, D), jnp.float32),
    ],
)

_sc_call = pl.kernel(
    _sc_body,
    out_type=[
        jax.ShapeDtypeStruct((NPAD, D), jnp.float32),
        jax.ShapeDtypeStruct((NPAD, D), jnp.float32),
    ],
    mesh=plsc.VectorSubcoreMesh(core_axis_name="c", subcore_axis_name="s"),
    scratch_types=[
        pltpu.VMEM((3, CHUNK), jnp.int32),
        pltpu.VMEM((3, CHUNK), jnp.int32),
        pltpu.VMEM((CHUNK,), jnp.int32),
        pltpu.VMEM((CHUNK,), jnp.int32),
        pltpu.VMEM((CHUNK, D), jnp.float32),
        pltpu.VMEM((CHUNK, D), jnp.float32),
        pltpu.VMEM((CHUNK, D), jnp.float32),
        pltpu.VMEM((CHUNK, D), jnp.float32),
        pltpu.VMEM((CHUNK, D), jnp.float32),
        pltpu.VMEM((CHUNK, D), jnp.float32),
        pltpu.VMEM_SHARED((NPAD, D), jnp.float32),
        pltpu.SemaphoreType.DMA,
        pltpu.SemaphoreType.DMA,
        pltpu.SemaphoreType.DMA,
        pltpu.SemaphoreType.DMA,
        pltpu.SemaphoreType.DMA,
        pltpu.SemaphoreType.DMA,
    ],
)


def kernel(x_chemical, x_gene, edge_index_cg, edge_index_gc, edge_attr_cg,
           edge_attr_gc, W_src_cg, b_src_cg, W_dst_cg, b_dst_cg, W_cat_cg,
           b_cat_cg, attn_cg, W_src_gc, b_src_gc, W_dst_gc, b_dst_gc,
           W_cat_gc, b_cat_gc, attn_gc, emb_action_type, emb_action_subject,
           W_out_chemical, b_out_chemical, W_out_gene, b_out_gene):
    num_as = emb_action_subject.shape[0]
    num_at = emb_action_type.shape[0]

    def _pack(edge_index, edge_attr):
        si = edge_index[0].astype(jnp.int32)
        di = edge_index[1].astype(jnp.int32)
        code = (edge_attr[:, 0] * num_as + edge_attr[:, 1]).astype(jnp.int32)
        pad = EPAD - si.shape[0]
        si = jnp.pad(si, (0, pad))
        di = jnp.pad(di, (0, pad))
        code = jnp.pad(code, (0, pad), constant_values=GATE_ROWS - 1)
        return jnp.stack([si.reshape(NCHUNKS, CHUNK),
                          di.reshape(NCHUNKS, CHUNK),
                          code.reshape(NCHUNKS, CHUNK)], axis=1)

    idx_cg = _pack(edge_index_cg, edge_attr_cg)
    idx_gc = _pack(edge_index_gc, edge_attr_gc)

    cat = jnp.concatenate(
        [jnp.repeat(emb_action_type, num_as, axis=0),
         jnp.tile(emb_action_subject, (num_at, 1))], axis=1)
    cat = jnp.pad(cat, ((0, GATE_ROWS - num_at * num_as), (0, 0)))

    xs_cg, xd_cg, xs_gc, xd_gc, g_cg, g_gc = _prep_call(
        x_chemical, x_gene,
        W_src_cg, b_src_cg.reshape(1, D), W_dst_cg, b_dst_cg.reshape(1, D),
        W_src_gc, b_src_gc.reshape(1, D), W_dst_gc, b_dst_gc.reshape(1, D),
        cat, W_cat_cg, b_cat_cg.reshape(1, D), W_cat_gc, b_cat_gc.reshape(1, D))

    aggr_gene_p, aggr_chem_p = _sc_call(
        xs_cg, xd_cg, g_cg, idx_cg,
        xs_gc, xd_gc, g_gc, idx_gc)
    aggr_gene = aggr_gene_p
    aggr_chem = aggr_chem_p

    out_chem, out_gene = _out_call(
        aggr_chem, W_out_chemical, b_out_chemical.reshape(1, D),
        aggr_gene, W_out_gene, b_out_gene.reshape(1, D))
    return (out_chem, out_gene)


# speedup vs baseline: 1.0373x; 1.0373x over previous
import functools

import jax
import jax.numpy as jnp
from jax import lax
from jax.experimental import pallas as pl
from jax.experimental.pallas import tpu as pltpu
from jax.experimental.pallas import tpu_sc as plsc

N_CHEM = 10000
N_GENE = 10000
E = 320000
D = 128
CHUNK = 48
NSUB = 16
NCHUNKS = 6672
N_PER_TILE = NCHUNKS // NSUB
EPAD = NCHUNKS * CHUNK
NPAD = 10000
ZONE = 640
LAST_ZONE = NPAD - 15 * ZONE
VALID_GATE = 50
GATE_ROWS = 64
NBLK = 10
BLK = N_CHEM // NBLK


def _sigmoid(x):
    return 1.0 / (1.0 + jnp.exp(-x))


def _prep_body(xc, xg, wscg, bscg, wdcg, bdcg, wsgc, bsgc, wdgc, bdgc,
               cat, wccg, bccg, wcgc, bcgc,
               xs_cg, xd_cg, xs_gc, xd_gc, g_cg, g_gc):
    c = xc[...]
    g = xg[...]
    xs_cg[...] = jnp.dot(c, wscg[...], preferred_element_type=jnp.float32) + bscg[...]
    xd_cg[...] = jnp.dot(g, wdcg[...], preferred_element_type=jnp.float32) + bdcg[...]
    xs_gc[...] = jnp.dot(g, wsgc[...], preferred_element_type=jnp.float32) + bsgc[...]
    xd_gc[...] = jnp.dot(c, wdgc[...], preferred_element_type=jnp.float32) + bdgc[...]

    @pl.when(pl.program_id(0) == 0)
    def _():
        t = cat[...]
        row = lax.broadcasted_iota(jnp.int32, (GATE_ROWS, D), 0)
        valid = row < VALID_GATE
        g_cg[...] = jnp.where(valid, _sigmoid(
            jnp.dot(t, wccg[...], preferred_element_type=jnp.float32)
            + bccg[...]) * 0.25, 0.0)
        g_gc[...] = jnp.where(valid, _sigmoid(
            jnp.dot(t, wcgc[...], preferred_element_type=jnp.float32)
            + bcgc[...]) * 0.25, 0.0)


def _out_body(ac, woc, boc, ag, wog, bog, oc, og):
    oc[...] = jnp.dot(ac[...], woc[...], preferred_element_type=jnp.float32) + boc[...]
    og[...] = jnp.dot(ag[...], wog[...], preferred_element_type=jnp.float32) + bog[...]


def _sc_body(xs0, xd0, g0, idx0h,
             xs1, xd1, g1, idx1h,
             out_g, out_c,
             idx_a, idx_b, di_sc_a, di_sc_b,
             src_a, dst_a, gate_a, msg_a, src_b, dst_b, gate_b, msg_b,
             acc, isem_a, isem_b, gsem_a, gsem_b, ssem_a, ssem_b):
    c = lax.axis_index("c")
    s = lax.axis_index("s")

    def _z(e, carry):
        for t in range(D // 16):
            msg_a[e, pl.ds(t * 16, 16)] = jnp.zeros((16,), jnp.float32)
        return carry
    lax.fori_loop(0, CHUNK, _z, 0)
    for k in range(8):
        pltpu.sync_copy(msg_a, acc.at[pl.ds(s * ZONE + k * CHUNK, CHUNK)])

    @pl.when(s < NSUB - 1)
    def _():
        for k in range(8, 13):
            pltpu.sync_copy(msg_a, acc.at[pl.ds(s * ZONE + k * CHUNK, CHUNK)])
        pltpu.sync_copy(msg_a.at[pl.ds(0, 16)],
                        acc.at[pl.ds(s * ZONE + 13 * CHUNK, 16)])

    @pl.when(s == NSUB - 1)
    def _():
        pltpu.sync_copy(msg_a.at[pl.ds(0, 16)],
                        acc.at[pl.ds(s * ZONE + 8 * CHUNK, 16)])
    plsc.subcore_barrier()

    def _process(xs, xd, gt, idxh):
        n = N_PER_TILE
        sets = (
            (idx_a, di_sc_a, src_a, dst_a, gate_a, msg_a, isem_a, gsem_a, ssem_a),
            (idx_b, di_sc_b, src_b, dst_b, gate_b, msg_b, isem_b, gsem_b, ssem_b),
        )

        def fire_idx(j, st):
            idx, isem = st[0], st[6]
            pltpu.async_copy(idxh.at[s + NSUB * j], idx, isem)

        def gather_stage(j, st):
            idx, di_sc, src, dst, gate, msg, isem, gsem, ssem = st
            pltpu.make_async_copy(idxh.at[s + NSUB * j], idx, isem).wait()
            pltpu.async_copy(xs.at[idx.at[0]], src, gsem)
            pltpu.async_copy(xd.at[idx.at[1]], dst, gsem)
            pltpu.async_copy(gt.at[idx.at[2]], gate, gsem)

        def compute_stage(j, st):
            idx, di_sc, src, dst, gate, msg, isem, gsem, ssem = st
            pltpu.make_async_copy(xs.at[idx.at[0]], src, gsem).wait()
            pltpu.make_async_copy(xd.at[idx.at[1]], dst, gsem).wait()
            pltpu.make_async_copy(gt.at[idx.at[2]], gate, gsem).wait()

            @pl.when(j >= 3)
            def _():
                pltpu.make_async_copy(msg, acc.at[di_sc], ssem).wait()

            for k in range(CHUNK // 16):
                sl = pl.ds(k * 16, 16)
                di_sc[sl] = idx[1, sl]

            @pl.when(j < n - 1)
            def _():
                fire_idx(j + 1, st)

            @plsc.parallel_loop(0, CHUNK, 1, unroll=2)
            def _(e):
                for t in range(D // 16):
                    sl = pl.ds(t * 16, 16)
                    msg[e, sl] = (src[e, sl] + dst[e, sl]) * gate[e, sl]

            pltpu.async_copy(msg, acc.at[di_sc], ssem, add=True)

        fire_idx(0, sets[0])
        fire_idx(1, sets[1])

        def iter_body(j, carry):
            @pl.when(j % 2 == 0)
            def _():
                @pl.when(j < n)
                def _():
                    gather_stage(j, sets[0])

                @pl.when(j >= 1)
                def _():
                    compute_stage(j, sets[1])

            @pl.when(j % 2 == 1)
            def _():
                @pl.when(j < n)
                def _():
                    gather_stage(j, sets[1])
                compute_stage(j, sets[0])
            return carry
        lax.fori_loop(0, n + 1, iter_body, 0)

        pltpu.make_async_copy(msg_b, acc.at[di_sc_b], ssem_b).wait()
        pltpu.make_async_copy(msg_a, acc.at[di_sc_a], ssem_a).wait()

    @pl.when(c == 0)
    def _():
        _process(xs0, xd0, g0, idx0h)

    @pl.when(c == 1)
    def _():
        _process(xs1, xd1, g1, idx1h)

    plsc.subcore_barrier()

    def _copy_out(dst):
        @pl.when(s < NSUB - 1)
        def _():
            pltpu.sync_copy(acc.at[pl.ds(s * ZONE, ZONE)],
                            dst.at[pl.ds(s * ZONE, ZONE)])

        @pl.when(s == NSUB - 1)
        def _():
            pltpu.sync_copy(acc.at[pl.ds(s * ZONE, LAST_ZONE)],
                            dst.at[pl.ds(s * ZONE, LAST_ZONE)])

    @pl.when(c == 0)
    def _():
        _copy_out(out_g)

    @pl.when(c == 1)
    def _():
        _copy_out(out_c)


_full128 = pl.BlockSpec((D, D), lambda i: (0, 0))
_full1x = pl.BlockSpec((1, D), lambda i: (0, 0))
_blk = pl.BlockSpec((BLK, D), lambda i: (i, 0))

_prep_call = pl.pallas_call(
    _prep_body,
    grid=(NBLK,),
    in_specs=[
        _blk, _blk,
        _full128, _full1x, _full128, _full1x,
        _full128, _full1x, _full128, _full1x,
        pl.BlockSpec((GATE_ROWS, GATE_ROWS), lambda i: (0, 0)),
        pl.BlockSpec((GATE_ROWS, D), lambda i: (0, 0)), _full1x,
        pl.BlockSpec((GATE_ROWS, D), lambda i: (0, 0)), _full1x,
    ],
    out_specs=[
        _blk, _blk, _blk, _blk,
        pl.BlockSpec((GATE_ROWS, D), lambda i: (0, 0)),
        pl.BlockSpec((GATE_ROWS, D), lambda i: (0, 0)),
    ],
    out_shape=[
        jax.ShapeDtypeStruct((N_CHEM, D), jnp.float32),
        jax.ShapeDtypeStruct((N_GENE, D), jnp.float32),
        jax.ShapeDtypeStruct((N_GENE, D), jnp.float32),
        jax.ShapeDtypeStruct((N_CHEM, D), jnp.float32),
        jax.ShapeDtypeStruct((GATE_ROWS, D), jnp.float32),
        jax.ShapeDtypeStruct((GATE_ROWS, D), jnp.float32),
    ],
)

_out_call = pl.pallas_call(
    _out_body,
    grid=(NBLK,),
    in_specs=[_blk, _full128, _full1x, _blk, _full128, _full1x],
    out_specs=[_blk, _blk],
    out_shape=[
        jax.ShapeDtypeStruct((N_CHEM, D), jnp.float32),
        jax.ShapeDtypeStruct((N_GENE, D), jnp.float32),
    ],
)

_sc_call = pl.kernel(
    _sc_body,
    out_type=[
        jax.ShapeDtypeStruct((NPAD, D), jnp.float32),
        jax.ShapeDtypeStruct((NPAD, D), jnp.float32),
    ],
    mesh=plsc.VectorSubcoreMesh(core_axis_name="c", subcore_axis_name="s"),
    scratch_types=[
        pltpu.VMEM((3, CHUNK), jnp.int32),
        pltpu.VMEM((3, CHUNK), jnp.int32),
        pltpu.VMEM((CHUNK,), jnp.int32),
        pltpu.VMEM((CHUNK,), jnp.int32),
        pltpu.VMEM((CHUNK, D), jnp.float32),
        pltpu.VMEM((CHUNK, D), jnp.float32),
        pltpu.VMEM((CHUNK, D), jnp.float32),
        pltpu.VMEM((CHUNK, D), jnp.float32),
        pltpu.VMEM((CHUNK, D), jnp.float32),
        pltpu.VMEM((CHUNK, D), jnp.float32),
        pltpu.VMEM((CHUNK, D), jnp.float32),
        pltpu.VMEM((CHUNK, D), jnp.float32),
        pltpu.VMEM_SHARED((NPAD, D), jnp.float32),
        pltpu.SemaphoreType.DMA,
        pltpu.SemaphoreType.DMA,
        pltpu.SemaphoreType.DMA,
        pltpu.SemaphoreType.DMA,
        pltpu.SemaphoreType.DMA,
        pltpu.SemaphoreType.DMA,
    ],
)


def kernel(x_chemical, x_gene, edge_index_cg, edge_index_gc, edge_attr_cg,
           edge_attr_gc, W_src_cg, b_src_cg, W_dst_cg, b_dst_cg, W_cat_cg,
           b_cat_cg, attn_cg, W_src_gc, b_src_gc, W_dst_gc, b_dst_gc,
           W_cat_gc, b_cat_gc, attn_gc, emb_action_type, emb_action_subject,
           W_out_chemical, b_out_chemical, W_out_gene, b_out_gene):
    num_as = emb_action_subject.shape[0]
    num_at = emb_action_type.shape[0]

    def _pack(edge_index, edge_attr):
        si = edge_index[0].astype(jnp.int32)
        di = edge_index[1].astype(jnp.int32)
        code = (edge_attr[:, 0] * num_as + edge_attr[:, 1]).astype(jnp.int32)
        pad = EPAD - si.shape[0]
        si = jnp.pad(si, (0, pad))
        di = jnp.pad(di, (0, pad))
        code = jnp.pad(code, (0, pad), constant_values=GATE_ROWS - 1)
        return jnp.stack([si.reshape(NCHUNKS, CHUNK),
                          di.reshape(NCHUNKS, CHUNK),
                          code.reshape(NCHUNKS, CHUNK)], axis=1)

    idx_cg = _pack(edge_index_cg, edge_attr_cg)
    idx_gc = _pack(edge_index_gc, edge_attr_gc)

    cat = jnp.concatenate(
        [jnp.repeat(emb_action_type, num_as, axis=0),
         jnp.tile(emb_action_subject, (num_at, 1))], axis=1)
    cat = jnp.pad(cat, ((0, GATE_ROWS - num_at * num_as), (0, 0)))

    xs_cg, xd_cg, xs_gc, xd_gc, g_cg, g_gc = _prep_call(
        x_chemical, x_gene,
        W_src_cg, b_src_cg.reshape(1, D), W_dst_cg, b_dst_cg.reshape(1, D),
        W_src_gc, b_src_gc.reshape(1, D), W_dst_gc, b_dst_gc.reshape(1, D),
        cat, W_cat_cg, b_cat_cg.reshape(1, D), W_cat_gc, b_cat_gc.reshape(1, D))

    aggr_gene_p, aggr_chem_p = _sc_call(
        xs_cg, xd_cg, g_cg, idx_cg,
        xs_gc, xd_gc, g_gc, idx_gc)
    aggr_gene = aggr_gene_p
    aggr_chem = aggr_chem_p

    out_chem, out_gene = _out_call(
        aggr_chem, W_out_chemical, b_out_chemical.reshape(1, D),
        aggr_gene, W_out_gene, b_out_gene.reshape(1, D))
    return (out_chem, out_gene)


# speedup vs baseline: 1.5542x; 1.4983x over previous
import functools

import jax
import jax.numpy as jnp
from jax import lax
from jax.experimental import pallas as pl
from jax.experimental.pallas import tpu as pltpu
from jax.experimental.pallas import tpu_sc as plsc

N_CHEM = 10000
N_GENE = 10000
E = 320000
D = 128
CHUNK = 64
NSUB = 16
NCHUNKS = 5024
N_PER_TILE = NCHUNKS // NSUB
EPAD = NCHUNKS * CHUNK
DW = D // 2
NPAD = 10000
ZONE = 640
LAST_ZONE = NPAD - 15 * ZONE
VALID_GATE = 50
GATE_ROWS = 64
NBLK = 10
BLK = N_CHEM // NBLK


def _sigmoid(x):
    return 1.0 / (1.0 + jnp.exp(-x))


def _prep_body(xc, xg, wscg, bscg, wdcg, bdcg, wsgc, bsgc, wdgc, bdgc,
               cat, wccg, bccg, wcgc, bcgc,
               xs_cg, xd_cg, xs_gc, xd_gc, g_cg, g_gc):
    def _pack32(x):
        u = lax.bitcast_convert_type(x, jnp.int32)
        r = (u + 0x7FFF + ((u >> 16) & 1)) >> 16
        lo = r[:, :DW] & 0xFFFF
        hi = r[:, DW:]
        return lo | (hi << 16)

    c = xc[...]
    g = xg[...]
    xs_cg[...] = _pack32(
        jnp.dot(c, wscg[...], preferred_element_type=jnp.float32) + bscg[...])
    xd_cg[...] = _pack32(
        jnp.dot(g, wdcg[...], preferred_element_type=jnp.float32) + bdcg[...])
    xs_gc[...] = _pack32(
        jnp.dot(g, wsgc[...], preferred_element_type=jnp.float32) + bsgc[...])
    xd_gc[...] = _pack32(
        jnp.dot(c, wdgc[...], preferred_element_type=jnp.float32) + bdgc[...])

    @pl.when(pl.program_id(0) == 0)
    def _():
        t = cat[...]
        row = lax.broadcasted_iota(jnp.int32, (GATE_ROWS, D), 0)
        valid = row < VALID_GATE
        g_cg[...] = _pack32(jnp.where(valid, _sigmoid(
            jnp.dot(t, wccg[...], preferred_element_type=jnp.float32)
            + bccg[...]) * 0.25, 0.0))
        g_gc[...] = _pack32(jnp.where(valid, _sigmoid(
            jnp.dot(t, wcgc[...], preferred_element_type=jnp.float32)
            + bcgc[...]) * 0.25, 0.0))


def _out_body(ac, woc, boc, ag, wog, bog, oc, og):
    oc[...] = jnp.dot(ac[...], woc[...], preferred_element_type=jnp.float32) + boc[...]
    og[...] = jnp.dot(ag[...], wog[...], preferred_element_type=jnp.float32) + bog[...]


def _sc_body(xs0, xd0, g0, idx0h,
             xs1, xd1, g1, idx1h,
             out_g, out_c,
             idx_a, idx_b, di_sc_a, di_sc_b,
             src_a, dst_a, gate_a, msg_a, src_b, dst_b, gate_b, msg_b,
             acc, isem_a, isem_b, gsem_a, gsem_b, ssem_a, ssem_b):
    c = lax.axis_index("c")
    s = lax.axis_index("s")

    def _z(e, carry):
        for t in range(D // 16):
            msg_a[e, pl.ds(t * 16, 16)] = jnp.zeros((16,), jnp.float32)
        return carry
    lax.fori_loop(0, CHUNK, _z, 0)
    for k in range(6):
        pltpu.sync_copy(msg_a, acc.at[pl.ds(s * ZONE + k * CHUNK, CHUNK)])

    @pl.when(s < NSUB - 1)
    def _():
        for k in range(6, 10):
            pltpu.sync_copy(msg_a, acc.at[pl.ds(s * ZONE + k * CHUNK, CHUNK)])

    @pl.when(s == NSUB - 1)
    def _():
        pltpu.sync_copy(msg_a.at[pl.ds(0, 16)],
                        acc.at[pl.ds(s * ZONE + 6 * CHUNK, 16)])
    plsc.subcore_barrier()

    def _process(xs, xd, gt, idxh):
        n = N_PER_TILE
        sets = (
            (idx_a, di_sc_a, src_a, dst_a, gate_a, msg_a, isem_a, gsem_a, ssem_a),
            (idx_b, di_sc_b, src_b, dst_b, gate_b, msg_b, isem_b, gsem_b, ssem_b),
        )

        def fire_idx(j, st):
            idx, isem = st[0], st[6]
            pltpu.async_copy(idxh.at[s + NSUB * j], idx, isem)

        def gather_stage(j, st):
            idx, di_sc, src, dst, gate, msg, isem, gsem, ssem = st
            pltpu.make_async_copy(idxh.at[s + NSUB * j], idx, isem).wait()
            pltpu.async_copy(xs.at[idx.at[0]], src, gsem)
            pltpu.async_copy(xd.at[idx.at[1]], dst, gsem)
            pltpu.async_copy(gt.at[idx.at[2]], gate, gsem)

        def compute_stage(j, st):
            idx, di_sc, src, dst, gate, msg, isem, gsem, ssem = st
            pltpu.make_async_copy(xs.at[idx.at[0]], src, gsem).wait()
            pltpu.make_async_copy(xd.at[idx.at[1]], dst, gsem).wait()
            pltpu.make_async_copy(gt.at[idx.at[2]], gate, gsem).wait()

            @pl.when(j >= 3)
            def _():
                pltpu.make_async_copy(msg, acc.at[di_sc], ssem).wait()

            for k in range(CHUNK // 16):
                sl = pl.ds(k * 16, 16)
                di_sc[sl] = idx[1, sl]

            @pl.when(j < n - 1)
            def _():
                fire_idx(j + 1, st)

            @plsc.parallel_loop(0, CHUNK, 1, unroll=2)
            def _(e):
                hi_mask = jnp.int32(-65536)
                for t in range(DW // 16):
                    sl = pl.ds(t * 16, 16)
                    sw = src[e, sl]
                    dw = dst[e, sl]
                    gw = gate[e, sl]
                    s_lo = lax.bitcast_convert_type(sw << 16, jnp.float32)
                    d_lo = lax.bitcast_convert_type(dw << 16, jnp.float32)
                    g_lo = lax.bitcast_convert_type(gw << 16, jnp.float32)
                    s_hi = lax.bitcast_convert_type(sw & hi_mask, jnp.float32)
                    d_hi = lax.bitcast_convert_type(dw & hi_mask, jnp.float32)
                    g_hi = lax.bitcast_convert_type(gw & hi_mask, jnp.float32)
                    msg[e, pl.ds(t * 16, 16)] = (s_lo + d_lo) * g_lo
                    msg[e, pl.ds(DW + t * 16, 16)] = (s_hi + d_hi) * g_hi

            pltpu.async_copy(msg, acc.at[di_sc], ssem, add=True)

        fire_idx(0, sets[0])
        fire_idx(1, sets[1])

        def iter_body(j, carry):
            @pl.when(j % 2 == 0)
            def _():
                @pl.when(j < n)
                def _():
                    gather_stage(j, sets[0])

                @pl.when(j >= 1)
                def _():
                    compute_stage(j, sets[1])

            @pl.when(j % 2 == 1)
            def _():
                @pl.when(j < n)
                def _():
                    gather_stage(j, sets[1])
                compute_stage(j, sets[0])
            return carry
        lax.fori_loop(0, n + 1, iter_body, 0)

        pltpu.make_async_copy(msg_b, acc.at[di_sc_b], ssem_b).wait()
        pltpu.make_async_copy(msg_a, acc.at[di_sc_a], ssem_a).wait()

    @pl.when(c == 0)
    def _():
        _process(xs0, xd0, g0, idx0h)

    @pl.when(c == 1)
    def _():
        _process(xs1, xd1, g1, idx1h)

    plsc.subcore_barrier()

    def _copy_out(dst):
        @pl.when(s < NSUB - 1)
        def _():
            pltpu.sync_copy(acc.at[pl.ds(s * ZONE, ZONE)],
                            dst.at[pl.ds(s * ZONE, ZONE)])

        @pl.when(s == NSUB - 1)
        def _():
            pltpu.sync_copy(acc.at[pl.ds(s * ZONE, LAST_ZONE)],
                            dst.at[pl.ds(s * ZONE, LAST_ZONE)])

    @pl.when(c == 0)
    def _():
        _copy_out(out_g)

    @pl.when(c == 1)
    def _():
        _copy_out(out_c)


_full128 = pl.BlockSpec((D, D), lambda i: (0, 0))
_full1x = pl.BlockSpec((1, D), lambda i: (0, 0))
_blk = pl.BlockSpec((BLK, D), lambda i: (i, 0))

_prep_call = pl.pallas_call(
    _prep_body,
    grid=(NBLK,),
    in_specs=[
        _blk, _blk,
        _full128, _full1x, _full128, _full1x,
        _full128, _full1x, _full128, _full1x,
        pl.BlockSpec((GATE_ROWS, GATE_ROWS), lambda i: (0, 0)),
        pl.BlockSpec((GATE_ROWS, D), lambda i: (0, 0)), _full1x,
        pl.BlockSpec((GATE_ROWS, D), lambda i: (0, 0)), _full1x,
    ],
    out_specs=[
        pl.BlockSpec((BLK, DW), lambda i: (i, 0)),
        pl.BlockSpec((BLK, DW), lambda i: (i, 0)),
        pl.BlockSpec((BLK, DW), lambda i: (i, 0)),
        pl.BlockSpec((BLK, DW), lambda i: (i, 0)),
        pl.BlockSpec((GATE_ROWS, DW), lambda i: (0, 0)),
        pl.BlockSpec((GATE_ROWS, DW), lambda i: (0, 0)),
    ],
    out_shape=[
        jax.ShapeDtypeStruct((N_CHEM, DW), jnp.int32),
        jax.ShapeDtypeStruct((N_GENE, DW), jnp.int32),
        jax.ShapeDtypeStruct((N_GENE, DW), jnp.int32),
        jax.ShapeDtypeStruct((N_CHEM, DW), jnp.int32),
        jax.ShapeDtypeStruct((GATE_ROWS, DW), jnp.int32),
        jax.ShapeDtypeStruct((GATE_ROWS, DW), jnp.int32),
    ],
)

_out_call = pl.pallas_call(
    _out_body,
    grid=(NBLK,),
    in_specs=[_blk, _full128, _full1x, _blk, _full128, _full1x],
    out_specs=[_blk, _blk],
    out_shape=[
        jax.ShapeDtypeStruct((N_CHEM, D), jnp.float32),
        jax.ShapeDtypeStruct((N_GENE, D), jnp.float32),
    ],
)

_sc_call = pl.kernel(
    _sc_body,
    out_type=[
        jax.ShapeDtypeStruct((NPAD, D), jnp.float32),
        jax.ShapeDtypeStruct((NPAD, D), jnp.float32),
    ],
    mesh=plsc.VectorSubcoreMesh(core_axis_name="c", subcore_axis_name="s"),
    compiler_params=pltpu.CompilerParams(use_tc_tiling_on_sc=False),
    scratch_types=[
        pltpu.VMEM((3, CHUNK), jnp.int32),
        pltpu.VMEM((3, CHUNK), jnp.int32),
        pltpu.VMEM((CHUNK,), jnp.int32),
        pltpu.VMEM((CHUNK,), jnp.int32),
        pltpu.VMEM((CHUNK, DW), jnp.int32),
        pltpu.VMEM((CHUNK, DW), jnp.int32),
        pltpu.VMEM((CHUNK, DW), jnp.int32),
        pltpu.VMEM((CHUNK, D), jnp.float32),
        pltpu.VMEM((CHUNK, DW), jnp.int32),
        pltpu.VMEM((CHUNK, DW), jnp.int32),
        pltpu.VMEM((CHUNK, DW), jnp.int32),
        pltpu.VMEM((CHUNK, D), jnp.float32),
        pltpu.VMEM_SHARED((NPAD, D), jnp.float32),
        pltpu.SemaphoreType.DMA,
        pltpu.SemaphoreType.DMA,
        pltpu.SemaphoreType.DMA,
        pltpu.SemaphoreType.DMA,
        pltpu.SemaphoreType.DMA,
        pltpu.SemaphoreType.DMA,
    ],
)


def kernel(x_chemical, x_gene, edge_index_cg, edge_index_gc, edge_attr_cg,
           edge_attr_gc, W_src_cg, b_src_cg, W_dst_cg, b_dst_cg, W_cat_cg,
           b_cat_cg, attn_cg, W_src_gc, b_src_gc, W_dst_gc, b_dst_gc,
           W_cat_gc, b_cat_gc, attn_gc, emb_action_type, emb_action_subject,
           W_out_chemical, b_out_chemical, W_out_gene, b_out_gene):
    num_as = emb_action_subject.shape[0]
    num_at = emb_action_type.shape[0]

    def _pack(edge_index, edge_attr):
        si = edge_index[0].astype(jnp.int32)
        di = edge_index[1].astype(jnp.int32)
        code = (edge_attr[:, 0] * num_as + edge_attr[:, 1]).astype(jnp.int32)
        pad = EPAD - si.shape[0]
        si = jnp.pad(si, (0, pad))
        di = jnp.pad(di, (0, pad))
        code = jnp.pad(code, (0, pad), constant_values=GATE_ROWS - 1)
        return jnp.stack([si.reshape(NCHUNKS, CHUNK),
                          di.reshape(NCHUNKS, CHUNK),
                          code.reshape(NCHUNKS, CHUNK)], axis=1)

    idx_cg = _pack(edge_index_cg, edge_attr_cg)
    idx_gc = _pack(edge_index_gc, edge_attr_gc)

    cat = jnp.concatenate(
        [jnp.repeat(emb_action_type, num_as, axis=0),
         jnp.tile(emb_action_subject, (num_at, 1))], axis=1)
    cat = jnp.pad(cat, ((0, GATE_ROWS - num_at * num_as), (0, 0)))

    xs_cg, xd_cg, xs_gc, xd_gc, g_cg, g_gc = _prep_call(
        x_chemical, x_gene,
        W_src_cg, b_src_cg.reshape(1, D), W_dst_cg, b_dst_cg.reshape(1, D),
        W_src_gc, b_src_gc.reshape(1, D), W_dst_gc, b_dst_gc.reshape(1, D),
        cat, W_cat_cg, b_cat_cg.reshape(1, D), W_cat_gc, b_cat_gc.reshape(1, D))

    aggr_gene_p, aggr_chem_p = _sc_call(
        xs_cg, xd_cg, g_cg, idx_cg,
        xs_gc, xd_gc, g_gc, idx_gc)
    aggr_gene = aggr_gene_p
    aggr_chem = aggr_chem_p

    out_chem, out_gene = _out_call(
        aggr_chem, W_out_chemical, b_out_chemical.reshape(1, D),
        aggr_gene, W_out_gene, b_out_gene.reshape(1, D))
    return (out_chem, out_gene)
